# baseline (speedup 1.0000x reference)
import functools

import jax
import jax.numpy as jnp
from jax import lax
from jax.experimental import pallas as pl
from jax.experimental.pallas import tpu as pltpu
from jax.experimental.pallas import tpu_sc as plsc

H = 512
W = 512
N = H * W
D = 20
K = 17 * 17 * 17 * 16
KC = 17 * 17 * 17
KP = 5120
NC = 2
NS = 16
NW = NC * NS
PPW = N // NW
L = 16
CH = KP // NW
CHH = CH // 2

_mesh = plsc.VectorSubcoreMesh(core_axis_name="c", subcore_axis_name="s")


@functools.partial(
    pl.kernel,
    mesh=_mesh,
    out_type=(
        jax.ShapeDtypeStruct((2 * NW * KP,), jnp.float32),
        jax.ShapeDtypeStruct((2 * KP, D), jnp.float32),
    ),
    compiler_params=pltpu.CompilerParams(
        needs_layout_passes=False, use_tc_tiling_on_sc=False),
    scratch_types=[
        pltpu.VMEM((PPW,), jnp.float32),
        pltpu.VMEM((PPW,), jnp.float32),
        pltpu.VMEM((PPW,), jnp.float32),
        pltpu.VMEM((PPW,), jnp.float32),
        pltpu.VMEM((PPW,), jnp.float32),
        pltpu.VMEM((PPW,), jnp.float32),
        pltpu.VMEM((KP,), jnp.float32),
        pltpu.VMEM((KP,), jnp.float32),
        pltpu.VMEM((L,), jnp.float32),
        pltpu.VMEM((CHH,), jnp.int32),
        pltpu.VMEM((CHH,), jnp.int32),
        pltpu.VMEM((CH, D), jnp.float32),
        pltpu.VMEM((CH, D), jnp.float32),
        pltpu.SemaphoreType.DMA,
    ],
)
def _hist_kernel(xin_hbm, xs_hbm, w_hbm, tm_hbm, tl_hbm,
                 hist_hbm, ctab_hbm,
                 a0, a1, a2, b0, b1, b2, hm, hl, wv,
                 idx_a, idx_b, rows_m, rows_l, sem):
    wid = lax.axis_index("s") * NC + lax.axis_index("c")
    base = wid * PPW
    k0 = wid * CH

    def idx_body(j, carry):
        half = j // (CHH // L)
        jj = j % (CHH // L)
        kk = k0 + half * CHH + jj * L + lax.iota(jnp.int32, L)
        v = jnp.minimum(kk, KC - 1) * 16

        @pl.when(half == 0)
        def _():
            idx_a[pl.ds(jj * L, L)] = v

        @pl.when(half == 1)
        def _():
            idx_b[pl.ds(jj * L, L)] = v

        return carry

    lax.fori_loop(0, CH // L, idx_body, 0)

    cps = (
        pltpu.async_copy(tm_hbm.at[idx_a], rows_m.at[pl.ds(0, CHH)], sem),
        pltpu.async_copy(tm_hbm.at[idx_b], rows_m.at[pl.ds(CHH, CHH)], sem),
        pltpu.async_copy(tl_hbm.at[idx_a], rows_l.at[pl.ds(0, CHH)], sem),
        pltpu.async_copy(tl_hbm.at[idx_b], rows_l.at[pl.ds(CHH, CHH)], sem),
    )

    pltpu.sync_copy(w_hbm, wv)
    pltpu.sync_copy(xin_hbm.at[pl.ds(base, PPW)], a0)
    pltpu.sync_copy(xin_hbm.at[pl.ds(N + base, PPW)], a1)
    pltpu.sync_copy(xin_hbm.at[pl.ds(2 * N + base, PPW)], a2)
    pltpu.sync_copy(xs_hbm.at[pl.ds(base, PPW)], b0)
    pltpu.sync_copy(xs_hbm.at[pl.ds(N + base, PPW)], b1)
    pltpu.sync_copy(xs_hbm.at[pl.ds(2 * N + base, PPW)], b2)

    def zero_body(i, carry):
        z = jnp.zeros((L,), jnp.float32)
        hm[pl.ds(i * L, L)] = z
        hl[pl.ds(i * L, L)] = z
        return carry

    lax.fori_loop(0, KP // L, zero_body, 0)

    wvec = wv[pl.ds(0, L)] * 0.0625
    w0 = wvec[0]
    w1 = wvec[1]
    w2 = wvec[2]
    ones = jnp.ones((L,), jnp.float32)

    def body(i, carry):
        o = i * L
        im = (a0[pl.ds(o, L)] * w0 + a1[pl.ds(o, L)] * w1
              + a2[pl.ds(o, L)] * w2).astype(jnp.int32)
        il = (b0[pl.ds(o, L)] * w0 + b1[pl.ds(o, L)] * w1
              + b2[pl.ds(o, L)] * w2).astype(jnp.int32)
        plsc.addupdate_scatter(hm, [im], ones)
        plsc.addupdate_scatter(hl, [il], ones)
        return carry

    lax.fori_loop(0, PPW // L, body, 0)

    pltpu.sync_copy(hm, hist_hbm.at[pl.ds(wid * KP, KP)])
    pltpu.sync_copy(hl, hist_hbm.at[pl.ds((NW + wid) * KP, KP)])

    for cp in cps:
        cp.wait()
    pltpu.sync_copy(rows_m, ctab_hbm.at[pl.ds(k0, CH)])
    pltpu.sync_copy(rows_l, ctab_hbm.at[pl.ds(KP + k0, CH)])


BK = 1024
NBLK = KP // BK


def _reduce_body(h_ref, tm_ref, tl_ref, o_ref, acc):
    i = pl.program_id(0)

    @pl.when(i == 0)
    def _init():
        acc[...] = jnp.zeros_like(acc)

    cm = jnp.sum(h_ref[:NW, :], axis=0).reshape(BK, 1)
    cl = jnp.sum(h_ref[NW:, :], axis=0).reshape(BK, 1)
    contrib = cm * tm_ref[...] + cl * tl_ref[...]
    acc[...] += jnp.sum(contrib, axis=0).reshape(1, D)

    @pl.when(i == NBLK - 1)
    def _done():
        r = acc[...] * (1.0 / N)
        o_ref[...] = jnp.clip(jnp.round(r * 4.0) * 0.25, -32.0, 31.75)


@jax.jit
def kernel(x_in, x_s, feature_msb, feature_lsb, weights):
    xin = x_in.reshape(3 * N)
    xs = x_s.reshape(3 * N)
    wpad = jnp.pad(weights.reshape(3).astype(jnp.float32), (0, L - 3))
    tm2 = feature_msb.reshape(K, D)
    tl2 = feature_lsb.reshape(K, D)

    hists_flat, ctab = _hist_kernel(xin, xs, wpad, tm2, tl2)
    hists = hists_flat.reshape(2 * NW, KP)
    ctabs = ctab.reshape(2, KP, D)

    out = pl.pallas_call(
        _reduce_body,
        grid=(NBLK,),
        in_specs=[
            pl.BlockSpec((2 * NW, BK), lambda i: (0, i)),
            pl.BlockSpec((BK, D), lambda i: (i, 0)),
            pl.BlockSpec((BK, D), lambda i: (i, 0)),
        ],
        out_specs=pl.BlockSpec((1, D), lambda i: (0, 0)),
        scratch_shapes=[pltpu.VMEM((1, D), jnp.float32)],
        out_shape=jax.ShapeDtypeStruct((1, D), jnp.float32),
    )(hists, ctabs[0], ctabs[1])
    return out.reshape(1, D, 1, 1)


# speedup vs baseline: 1.5108x; 1.5108x over previous
import functools

import jax
import jax.numpy as jnp
from jax import lax
from jax.experimental import pallas as pl
from jax.experimental.pallas import tpu as pltpu
from jax.experimental.pallas import tpu_sc as plsc

H = 512
W = 512
N = H * W
D = 20
KC = 17 * 17 * 17
KP = 4992
NC = 2
NS = 16
NW = NC * NS
PPW = N // NW
L = 16

_mesh = plsc.VectorSubcoreMesh(core_axis_name="c", subcore_axis_name="s")


@functools.partial(
    pl.kernel,
    mesh=_mesh,
    out_type=jax.ShapeDtypeStruct((2 * NW * KP,), jnp.float32),
    compiler_params=pltpu.CompilerParams(
        needs_layout_passes=False, use_tc_tiling_on_sc=False),
    scratch_types=[
        pltpu.VMEM((PPW,), jnp.float32),
        pltpu.VMEM((PPW,), jnp.float32),
        pltpu.VMEM((PPW,), jnp.float32),
        pltpu.VMEM((PPW,), jnp.float32),
        pltpu.VMEM((PPW,), jnp.float32),
        pltpu.VMEM((PPW,), jnp.float32),
        pltpu.VMEM((KP,), jnp.float32),
        pltpu.VMEM((KP,), jnp.float32),
        pltpu.VMEM((L,), jnp.float32),
    ],
)
def _hist_kernel(xin_hbm, xs_hbm, w_hbm, out_hbm,
                 a0, a1, a2, b0, b1, b2, hm, hl, wv):
    wid = lax.axis_index("s") * NC + lax.axis_index("c")
    base = wid * PPW

    pltpu.sync_copy(w_hbm, wv)
    pltpu.sync_copy(xin_hbm.at[pl.ds(base, PPW)], a0)
    pltpu.sync_copy(xin_hbm.at[pl.ds(N + base, PPW)], a1)
    pltpu.sync_copy(xin_hbm.at[pl.ds(2 * N + base, PPW)], a2)
    pltpu.sync_copy(xs_hbm.at[pl.ds(base, PPW)], b0)
    pltpu.sync_copy(xs_hbm.at[pl.ds(N + base, PPW)], b1)
    pltpu.sync_copy(xs_hbm.at[pl.ds(2 * N + base, PPW)], b2)

    def zero_body(i, carry):
        z = jnp.zeros((L,), jnp.float32)
        hm[pl.ds(i * L, L)] = z
        hl[pl.ds(i * L, L)] = z
        return carry

    lax.fori_loop(0, KP // L, zero_body, 0)

    wvec = wv[pl.ds(0, L)] * 0.0625
    w0 = wvec[0]
    w1 = wvec[1]
    w2 = wvec[2]
    ones = jnp.ones((L,), jnp.float32)

    def body(i, carry):
        o = i * L
        im = (a0[pl.ds(o, L)] * w0 + a1[pl.ds(o, L)] * w1
              + a2[pl.ds(o, L)] * w2).astype(jnp.int32)
        il = (b0[pl.ds(o, L)] * w0 + b1[pl.ds(o, L)] * w1
              + b2[pl.ds(o, L)] * w2).astype(jnp.int32)
        plsc.addupdate_scatter(hm, [im], ones)
        plsc.addupdate_scatter(hl, [il], ones)
        return carry

    lax.fori_loop(0, PPW // L, body, 0)

    pltpu.sync_copy(hm, out_hbm.at[pl.ds(wid * KP, KP)])
    pltpu.sync_copy(hl, out_hbm.at[pl.ds((NW + wid) * KP, KP)])


def _reduce_body(h_ref, tm_ref, tl_ref, o_ref):
    cm = jnp.sum(h_ref[:NW, :], axis=0, keepdims=True)
    cl = jnp.sum(h_ref[NW:, :], axis=0, keepdims=True)
    s = (jnp.dot(cm, tm_ref[...], precision=lax.Precision.HIGHEST,
                 preferred_element_type=jnp.float32)
         + jnp.dot(cl, tl_ref[...], precision=lax.Precision.HIGHEST,
                   preferred_element_type=jnp.float32))
    r = s * (1.0 / N)
    o_ref[...] = jnp.clip(jnp.round(r * 4.0) * 0.25, -32.0, 31.75)


@jax.jit
def kernel(x_in, x_s, feature_msb, feature_lsb, weights):
    xin = x_in.reshape(3 * N)
    xs = x_s.reshape(3 * N)
    wpad = jnp.pad(weights.reshape(3).astype(jnp.float32), (0, L - 3))

    hists = _hist_kernel(xin, xs, wpad).reshape(2 * NW, KP)

    ridx = jnp.arange(KP, dtype=jnp.int32) * 16
    ridx = jnp.minimum(ridx, 16 * (KC - 1))
    tm = jnp.take(feature_msb.reshape(-1, D), ridx, axis=0)
    tl = jnp.take(feature_lsb.reshape(-1, D), ridx, axis=0)

    out = pl.pallas_call(
        _reduce_body,
        out_shape=jax.ShapeDtypeStruct((1, D), jnp.float32),
    )(hists, tm, tl)
    return out.reshape(1, D, 1, 1)


# speedup vs baseline: 1.5386x; 1.0184x over previous
import functools

import jax
import jax.numpy as jnp
from jax import lax
from jax.experimental import pallas as pl
from jax.experimental.pallas import tpu as pltpu
from jax.experimental.pallas import tpu_sc as plsc

H = 512
W = 512
N = H * W
D = 20
KC = 17 * 17 * 17
KP = 4992
NC = 2
NS = 16
NW = NC * NS
PPW = N // NW
L = 16

_mesh = plsc.VectorSubcoreMesh(core_axis_name="c", subcore_axis_name="s")


@functools.partial(
    pl.kernel,
    mesh=_mesh,
    out_type=jax.ShapeDtypeStruct((2 * NW * KP,), jnp.float32),
    compiler_params=pltpu.CompilerParams(
        needs_layout_passes=False, use_tc_tiling_on_sc=False),
    scratch_types=[
        pltpu.VMEM((PPW,), jnp.float32),
        pltpu.VMEM((PPW,), jnp.float32),
        pltpu.VMEM((PPW,), jnp.float32),
        pltpu.VMEM((PPW,), jnp.float32),
        pltpu.VMEM((PPW,), jnp.float32),
        pltpu.VMEM((PPW,), jnp.float32),
        pltpu.VMEM((KP,), jnp.float32),
        pltpu.VMEM((KP,), jnp.float32),
        pltpu.VMEM((L,), jnp.float32),
    ],
)
def _hist_kernel(xin_hbm, xs_hbm, w_hbm, out_hbm,
                 a0, a1, a2, b0, b1, b2, hm, hl, wv):
    wid = lax.axis_index("s") * NC + lax.axis_index("c")
    base = wid * PPW

    pltpu.sync_copy(w_hbm, wv)
    pltpu.sync_copy(xin_hbm.at[pl.ds(base, PPW)], a0)
    pltpu.sync_copy(xin_hbm.at[pl.ds(N + base, PPW)], a1)
    pltpu.sync_copy(xin_hbm.at[pl.ds(2 * N + base, PPW)], a2)
    pltpu.sync_copy(xs_hbm.at[pl.ds(base, PPW)], b0)
    pltpu.sync_copy(xs_hbm.at[pl.ds(N + base, PPW)], b1)
    pltpu.sync_copy(xs_hbm.at[pl.ds(2 * N + base, PPW)], b2)

    def zero_body(i, carry):
        z = jnp.zeros((L,), jnp.float32)
        hm[pl.ds(i * L, L)] = z
        hl[pl.ds(i * L, L)] = z
        return carry

    lax.fori_loop(0, KP // L, zero_body, 0)

    wvec = wv[pl.ds(0, L)] * 0.0625
    w0 = wvec[0]
    w1 = wvec[1]
    w2 = wvec[2]
    ones = jnp.ones((L,), jnp.float32)

    def body(i, carry):
        o = i * L
        im = (a0[pl.ds(o, L)] * w0 + a1[pl.ds(o, L)] * w1
              + a2[pl.ds(o, L)] * w2).astype(jnp.int32)
        il = (b0[pl.ds(o, L)] * w0 + b1[pl.ds(o, L)] * w1
              + b2[pl.ds(o, L)] * w2).astype(jnp.int32)
        plsc.addupdate_scatter(hm, [im], ones)
        plsc.addupdate_scatter(hl, [il], ones)
        return carry

    lax.fori_loop(0, PPW // L, body, 0)

    pltpu.sync_copy(hm, out_hbm.at[pl.ds(wid * KP, KP)])
    pltpu.sync_copy(hl, out_hbm.at[pl.ds((NW + wid) * KP, KP)])


def _reduce_body(h_ref, tm_ref, tl_ref, o_ref):
    cm = jnp.sum(h_ref[:NW, :], axis=0, keepdims=True)
    cl = jnp.sum(h_ref[NW:, :], axis=0, keepdims=True)
    s = (jnp.dot(cm, tm_ref[...], precision=lax.Precision.HIGHEST,
                 preferred_element_type=jnp.float32)
         + jnp.dot(cl, tl_ref[...], precision=lax.Precision.HIGHEST,
                   preferred_element_type=jnp.float32))
    r = s * (1.0 / N)
    o_ref[...] = jnp.clip(jnp.round(r * 4.0) * 0.25, -32.0, 31.75)


@jax.jit
def kernel(x_in, x_s, feature_msb, feature_lsb, weights):
    xin = x_in.reshape(3 * N)
    xs = x_s.reshape(3 * N)
    wpad = jnp.pad(weights.reshape(3).astype(jnp.float32), (0, L - 3))

    hists = _hist_kernel(xin, xs, wpad).reshape(2 * NW, KP)

    tm = jnp.pad(feature_msb.reshape(KC, 16 * D)[:, :D], ((0, KP - KC), (0, 0)))
    tl = jnp.pad(feature_lsb.reshape(KC, 16 * D)[:, :D], ((0, KP - KC), (0, 0)))

    out = pl.pallas_call(
        _reduce_body,
        out_shape=jax.ShapeDtypeStruct((1, D), jnp.float32),
    )(hists, tm, tl)
    return out.reshape(1, D, 1, 1)
